# baseline (speedup 1.0000x reference)
import jax
import jax.numpy as jnp
from jax import lax
from jax.experimental import pallas as pl
from jax.experimental.pallas import tpu as pltpu
from jax.experimental.pallas import tpu_sc as plsc

N = 50000
E = 800000
HALF = 25000
ACC = 25400
SROWS = 2 * ACC
NC, NS = 2, 16
EROWS = 6272
EPAD = EROWS * 128
RPT = EROWS // NS
CB = 8
NBLK = RPT // CB
CAP = 1280
TPT = 1584
RBLK = 200


def _fill_rows(rows, nrows, d, vec):
    def zb(i, _):
        rows[i // (d // 16), pl.ds((i % (d // 16)) * 16, 16)] = vec
        return 0

    lax.fori_loop(0, nrows * (d // 16), zb, 0)


def _zero_acc(rows, acc, sid):
    for t in range(6):
        pltpu.sync_copy(rows.at[pl.ds(0, 264)],
                        acc.at[pl.ds(sid * TPT + t * 264, 264)])


def _drain_acc(out_hbm, acc, c, sid):
    pltpu.sync_copy(acc.at[pl.ds(sid * TPT, TPT)],
                    out_hbm.at[pl.ds(c * ACC + sid * TPT, TPT)])


def _compact_block(sidx, didx, csrc, cdst, base, woff):
    sentinel = jnp.int32(0x7FFFFFFF)

    def cv(i, w):
        j = i // 8
        k = i % 8
        dvec = didx[j, pl.ds(k * 16, 16)]
        l = dvec - base
        ok = (l >= 0) & (l < HALF)
        if sidx is not None:
            svec = sidx[j, pl.ds(k * 16, 16)]
            combo = jnp.where(ok, l * 65536 + svec, sentinel)
            sc = lax.sort(combo)
            cdst[pl.ds(w, 16)] = lax.shift_right_logical(sc, 16)
            csrc[pl.ds(w, 16)] = jnp.bitwise_and(sc, 0xFFFF)
        else:
            cdst[pl.ds(w, 16)] = lax.sort(jnp.where(ok, l, sentinel))
        return w + jnp.sum(ok.astype(jnp.int32))

    return lax.fori_loop(0, CB * 8, cv, woff)


def _carry_remainder(bufs, woff, ngroups):
    @pl.when(ngroups > 0)
    def _():
        off = ngroups * 128
        for buf in bufs:
            for k in range(8):
                v = buf[pl.ds(off + k * 16, 16)]
                buf[pl.ds(k * 16, 16)] = v

    return woff - ngroups * 128


def _make_scatter(d, nb, sd):
    mesh = plsc.VectorSubcoreMesh(core_axis_name="c", subcore_axis_name="s",
                                  num_cores=NC, num_subcores=NS)

    def body(g_hbm, src_hbm, dst_hbm, out_hbm, sidx, didx, csrc, cdst, rows,
             acc, sem_g, sem_s):
        c = lax.axis_index("c")
        sid = lax.axis_index("s")
        base = c * HALF
        trash = HALF + sid * 16 + lax.iota(jnp.int32, 16)
        ebase = sid * RPT
        _fill_rows(rows, 264, d, jnp.zeros((16,), jnp.float32))
        _zero_acc(rows, acc, sid)
        plsc.subcore_barrier()

        def wait_gather():
            pltpu.make_async_copy(
                g_hbm.at[csrc.at[pl.ds(0, 128)]],
                rows.at[pl.ds(0, 128)], sem_g).wait()

        def wait_scatter():
            pltpu.make_async_copy(
                rows.at[pl.ds(0, 128)],
                acc.at[cdst.at[pl.ds(0, 128)]], sem_s).wait()

        def issue_gather(g):
            pltpu.async_copy(
                g_hbm.at[csrc.at[pl.ds(g * 128, 128)]],
                rows.at[pl.ds(lax.rem(g, nb) * 128, 128)], sem_g)

        def issue_scatter(g):
            pltpu.async_copy(
                rows.at[pl.ds(lax.rem(g, nb) * 128, 128)],
                acc.at[cdst.at[pl.ds(g * 128, 128)]], sem_s, add=True)

        def stream_groups(ngroups):
            for k in range(nb - 1):
                @pl.when(k < ngroups)
                def _():
                    issue_gather(k)

            def grp(g, _):
                wait_gather()
                issue_scatter(g)

                @pl.when(g >= sd)
                def _():
                    wait_scatter()

                @pl.when(g + nb - 1 < ngroups)
                def _():
                    issue_gather(g + nb - 1)

                return 0

            lax.fori_loop(0, ngroups, grp, 0)
            for k in range(sd):
                @pl.when(k < ngroups)
                def _():
                    wait_scatter()

        def blk(b, woff):
            r0 = ebase + b * CB
            pltpu.sync_copy(src_hbm.at[pl.ds(r0, CB)], sidx)
            pltpu.sync_copy(dst_hbm.at[pl.ds(r0, CB)], didx)
            woff = _compact_block(sidx, didx, csrc, cdst, base, woff)
            ngroups = woff // 128
            stream_groups(ngroups)
            return _carry_remainder((csrc, cdst), woff, ngroups)

        rem = lax.fori_loop(0, NBLK, blk, 0)
        for k in range(8):
            cdst[pl.ds(rem + k * 16, 16)] = trash
            csrc[pl.ds(rem + k * 16, 16)] = jnp.zeros((16,), jnp.int32)

        @pl.when(rem > 0)
        def _():
            issue_gather(0)
            wait_gather()
            issue_scatter(0)
            wait_scatter()

        plsc.subcore_barrier()
        _drain_acc(out_hbm, acc, c, sid)

    return pl.kernel(
        body,
        out_type=jax.ShapeDtypeStruct((SROWS, d), jnp.float32),
        mesh=mesh,
        compiler_params=pltpu.CompilerParams(use_tc_tiling_on_sc=False,
                                             needs_layout_passes=False),
        scratch_types=[
            pltpu.VMEM((CB, 128), jnp.int32),
            pltpu.VMEM((CB, 128), jnp.int32),
            pltpu.VMEM((CAP,), jnp.int32),
            pltpu.VMEM((CAP,), jnp.int32),
            pltpu.VMEM((nb * 128, d), jnp.float32),
            pltpu.VMEM_SHARED((ACC, d), jnp.float32),
            pltpu.SemaphoreType.DMA,
            pltpu.SemaphoreType.DMA,
        ],
    )


def _make_deg(sd):
    mesh = plsc.VectorSubcoreMesh(core_axis_name="c", subcore_axis_name="s",
                                  num_cores=NC, num_subcores=NS)
    d = 16

    def body(dst_hbm, out_hbm, didx, cdst, rows, acc, sem_s):
        c = lax.axis_index("c")
        sid = lax.axis_index("s")
        base = c * HALF
        trash = HALF + sid * 16 + lax.iota(jnp.int32, 16)
        ebase = sid * RPT
        _fill_rows(rows, 264, d, jnp.zeros((16,), jnp.float32))
        _zero_acc(rows, acc, sid)
        _fill_rows(rows, 128, d,
                   jnp.where(lax.iota(jnp.int32, 16) == 0, 1.0, 0.0))
        plsc.subcore_barrier()

        def wait_scatter():
            pltpu.make_async_copy(
                rows.at[pl.ds(0, 128)],
                acc.at[cdst.at[pl.ds(0, 128)]], sem_s).wait()

        def issue_scatter(g):
            pltpu.async_copy(
                rows.at[pl.ds(0, 128)],
                acc.at[cdst.at[pl.ds(g * 128, 128)]], sem_s, add=True)

        def blk(b, woff):
            r0 = ebase + b * CB
            pltpu.sync_copy(dst_hbm.at[pl.ds(r0, CB)], didx)
            woff = _compact_block(None, didx, None, cdst, base, woff)
            ngroups = woff // 128

            def grp(g, _):
                issue_scatter(g)

                @pl.when(g >= sd)
                def _():
                    wait_scatter()

                return 0

            lax.fori_loop(0, ngroups, grp, 0)
            for k in range(sd):
                @pl.when(k < ngroups)
                def _():
                    wait_scatter()

            return _carry_remainder((cdst,), woff, ngroups)

        rem = lax.fori_loop(0, NBLK, blk, 0)
        for k in range(8):
            cdst[pl.ds(rem + k * 16, 16)] = trash

        @pl.when(rem > 0)
        def _():
            issue_scatter(0)
            wait_scatter()

        plsc.subcore_barrier()
        _drain_acc(out_hbm, acc, c, sid)

    return pl.kernel(
        body,
        out_type=jax.ShapeDtypeStruct((SROWS, d), jnp.float32),
        mesh=mesh,
        compiler_params=pltpu.CompilerParams(use_tc_tiling_on_sc=False,
                                             needs_layout_passes=False),
        scratch_types=[
            pltpu.VMEM((CB, 128), jnp.int32),
            pltpu.VMEM((CAP,), jnp.int32),
            pltpu.VMEM((264, d), jnp.float32),
            pltpu.VMEM_SHARED((ACC, d), jnp.float32),
            pltpu.SemaphoreType.DMA,
        ],
    )


_scat16 = _make_scatter(16, 8, 4)
_scat64 = _make_scatter(64, 3, 1)
_deg = _make_deg(4)


def _sblk(i):
    return ((i // 125) * 127 + (i % 125), 0)


def _tc_prep(deg16, x):
    def body(deg_ref, x_ref, g_ref):
        dinv = lax.rsqrt(deg_ref[:, 0:1] + 1.0)
        gx = x_ref[...] * dinv
        g_ref[...] = jnp.concatenate(
            [gx, jnp.zeros((RBLK, 16 - gx.shape[1]), jnp.float32)], axis=1)

    return pl.pallas_call(
        body,
        grid=(N // RBLK,),
        in_specs=[
            pl.BlockSpec((RBLK, 16), _sblk),
            pl.BlockSpec((RBLK, 5), lambda i: (i, 0)),
        ],
        out_specs=pl.BlockSpec((RBLK, 16), lambda i: (i, 0)),
        out_shape=jax.ShapeDtypeStruct((N, 16), jnp.float32),
    )(deg16, x)


def _tc_layer(s, g, deg16, w, b, relu_scale):
    din, dout = w.shape

    def body(s_ref, g_ref, deg_ref, w_ref, b_ref, o_ref):
        dinv = lax.rsqrt(deg_ref[:, 0:1] + 1.0)
        p = (s_ref[...] + g_ref[...]) * dinv
        h = jnp.dot(p, w_ref[...],
                    preferred_element_type=jnp.float32) + b_ref[...]
        if relu_scale:
            o_ref[...] = jnp.maximum(h, 0.0) * dinv
        else:
            o_ref[...] = h

    return pl.pallas_call(
        body,
        grid=(N // RBLK,),
        in_specs=[
            pl.BlockSpec((RBLK, din), _sblk),
            pl.BlockSpec((RBLK, din), lambda i: (i, 0)),
            pl.BlockSpec((RBLK, 16), _sblk),
            pl.BlockSpec((din, dout), lambda i: (0, 0)),
            pl.BlockSpec((1, dout), lambda i: (0, 0)),
        ],
        out_specs=pl.BlockSpec((RBLK, dout), lambda i: (i, 0)),
        out_shape=jax.ShapeDtypeStruct((N, dout), jnp.float32),
    )(s, g, deg16, w, b)


def kernel(x, edge_index, W1, b1, W2, b2, W3, b3):
    ei = edge_index.astype(jnp.int32)
    pad = EPAD - E
    srcp = jnp.concatenate(
        [ei[0], jnp.zeros((pad,), jnp.int32)]).reshape(EROWS, 128)
    dstp = jnp.concatenate(
        [ei[1], jnp.full((pad,), 2**30, jnp.int32)]).reshape(EROWS, 128)

    deg16 = _deg(dstp)
    g1 = _tc_prep(deg16, x)
    s1 = _scat16(g1, srcp, dstp)
    w1p = jnp.concatenate([W1, jnp.zeros((11, 64), jnp.float32)], axis=0)
    g2 = _tc_layer(s1, g1, deg16, w1p, b1.reshape(1, -1), True)
    s2 = _scat64(g2, srcp, dstp)
    g3 = _tc_layer(s2, g2, deg16, W2, b2.reshape(1, -1), True)
    s3 = _scat64(g3, srcp, dstp)
    return _tc_layer(s3, g3, deg16, W3, b3.reshape(1, -1), False)


# speedup vs baseline: 1.2155x; 1.2155x over previous
import jax
import jax.numpy as jnp
from jax import lax
from jax.experimental import pallas as pl
from jax.experimental.pallas import tpu as pltpu
from jax.experimental.pallas import tpu_sc as plsc

N = 50000
E = 800000
HALF = 25000
ACC = 26000
SROWS = 2 * ACC
NC, NS = 2, 16
EROWS = 6272
EPAD = EROWS * 128
RPT = EROWS // NS
CB = 8
NBLK = RPT // CB
CAP = 1280
TPT = 1624
RBLK = 1000


def _fill_rows(rows, nrows, d, vec):
    def zb(i, _):
        rows[i // (d // 16), pl.ds((i % (d // 16)) * 16, 16)] = vec
        return 0

    lax.fori_loop(0, nrows * (d // 16), zb, 0)


def _zero_acc(rows, acc, sid):
    for t in range(7):
        pltpu.sync_copy(rows.at[pl.ds(0, 232)],
                        acc.at[pl.ds(sid * TPT + t * 232, 232)])


def _drain_acc(out_hbm, acc, c, sid):
    pltpu.sync_copy(acc.at[pl.ds(sid * TPT, TPT)],
                    out_hbm.at[pl.ds(c * ACC + sid * TPT, TPT)])


def _compact_block(sidx, didx, csrc, cdst, base, woff):
    sentinel = jnp.int32(0x7FFFFFFF)

    def cv(i, w):
        j = i // 8
        k = i % 8
        dvec = didx[j, pl.ds(k * 16, 16)]
        l = dvec - base
        ok = (l >= 0) & (l < HALF)
        if sidx is not None:
            svec = sidx[j, pl.ds(k * 16, 16)]
            combo = jnp.where(ok, l * 65536 + svec, sentinel)
            sc = lax.sort(combo)
            cdst[pl.ds(w, 16)] = lax.shift_right_logical(sc, 16)
            csrc[pl.ds(w, 16)] = jnp.bitwise_and(sc, 0xFFFF)
        else:
            cdst[pl.ds(w, 16)] = lax.sort(jnp.where(ok, l, sentinel))
        return w + jnp.sum(ok.astype(jnp.int32))

    return lax.fori_loop(0, CB * 8, cv, woff)


def _carry_remainder(bufs, woff, ngroups):
    @pl.when(ngroups > 0)
    def _():
        off = ngroups * 128
        for buf in bufs:
            for k in range(8):
                v = buf[pl.ds(off + k * 16, 16)]
                buf[pl.ds(k * 16, 16)] = v

    return woff - ngroups * 128


def _make_scatter(d, nb, sd):
    mesh = plsc.VectorSubcoreMesh(core_axis_name="c", subcore_axis_name="s",
                                  num_cores=NC, num_subcores=NS)

    def body(g_hbm, src_hbm, dst_hbm, out_hbm, sidx, didx, csrc, cdst, rows,
             acc, sem_g, sem_s):
        c = lax.axis_index("c")
        sid = lax.axis_index("s")
        base = c * HALF
        trash = HALF + sid * 16 + lax.iota(jnp.int32, 16)
        ebase = sid * RPT
        _fill_rows(rows, 232, d, jnp.zeros((16,), jnp.float32))
        _zero_acc(rows, acc, sid)
        plsc.subcore_barrier()

        def wait_gather():
            pltpu.make_async_copy(
                g_hbm.at[csrc.at[pl.ds(0, 128)]],
                rows.at[pl.ds(0, 128)], sem_g).wait()

        def wait_scatter():
            pltpu.make_async_copy(
                rows.at[pl.ds(0, 128)],
                acc.at[cdst.at[pl.ds(0, 128)]], sem_s).wait()

        def issue_gather(g):
            pltpu.async_copy(
                g_hbm.at[csrc.at[pl.ds(g * 128, 128)]],
                rows.at[pl.ds(lax.rem(g, nb) * 128, 128)], sem_g)

        def issue_scatter(g):
            pltpu.async_copy(
                rows.at[pl.ds(lax.rem(g, nb) * 128, 128)],
                acc.at[cdst.at[pl.ds(g * 128, 128)]], sem_s, add=True)

        def stream_groups(ngroups):
            for k in range(nb - 1):
                @pl.when(k < ngroups)
                def _():
                    issue_gather(k)

            def grp(g, _):
                wait_gather()
                issue_scatter(g)

                @pl.when(g >= sd)
                def _():
                    wait_scatter()

                @pl.when(g + nb - 1 < ngroups)
                def _():
                    issue_gather(g + nb - 1)

                return 0

            lax.fori_loop(0, ngroups, grp, 0)
            for k in range(sd):
                @pl.when(k < ngroups)
                def _():
                    wait_scatter()

        def blk(b, woff):
            r0 = ebase + b * CB
            pltpu.sync_copy(src_hbm.at[pl.ds(r0, CB)], sidx)
            pltpu.sync_copy(dst_hbm.at[pl.ds(r0, CB)], didx)
            woff = _compact_block(sidx, didx, csrc, cdst, base, woff)
            ngroups = woff // 128
            stream_groups(ngroups)
            return _carry_remainder((csrc, cdst), woff, ngroups)

        rem = lax.fori_loop(0, NBLK, blk, 0)
        for k in range(8):
            cdst[pl.ds(rem + k * 16, 16)] = trash
            csrc[pl.ds(rem + k * 16, 16)] = jnp.zeros((16,), jnp.int32)

        @pl.when(rem > 0)
        def _():
            issue_gather(0)
            wait_gather()
            issue_scatter(0)
            wait_scatter()

        plsc.subcore_barrier()
        _drain_acc(out_hbm, acc, c, sid)

    return pl.kernel(
        body,
        out_type=jax.ShapeDtypeStruct((SROWS, d), jnp.float32),
        mesh=mesh,
        compiler_params=pltpu.CompilerParams(use_tc_tiling_on_sc=False,
                                             needs_layout_passes=False),
        scratch_types=[
            pltpu.VMEM((CB, 128), jnp.int32),
            pltpu.VMEM((CB, 128), jnp.int32),
            pltpu.VMEM((CAP,), jnp.int32),
            pltpu.VMEM((CAP,), jnp.int32),
            pltpu.VMEM((nb * 128, d), jnp.float32),
            pltpu.VMEM_SHARED((ACC, d), jnp.float32),
            pltpu.SemaphoreType.DMA,
            pltpu.SemaphoreType.DMA,
        ],
    )


def _make_deg(sd):
    mesh = plsc.VectorSubcoreMesh(core_axis_name="c", subcore_axis_name="s",
                                  num_cores=NC, num_subcores=NS)
    d = 16

    def body(dst_hbm, out_hbm, didx, cdst, rows, acc, sem_s):
        c = lax.axis_index("c")
        sid = lax.axis_index("s")
        base = c * HALF
        trash = HALF + sid * 16 + lax.iota(jnp.int32, 16)
        ebase = sid * RPT
        _fill_rows(rows, 232, d, jnp.zeros((16,), jnp.float32))
        _zero_acc(rows, acc, sid)
        _fill_rows(rows, 128, d,
                   jnp.where(lax.iota(jnp.int32, 16) == 0, 1.0, 0.0))
        plsc.subcore_barrier()

        def wait_scatter():
            pltpu.make_async_copy(
                rows.at[pl.ds(0, 128)],
                acc.at[cdst.at[pl.ds(0, 128)]], sem_s).wait()

        def issue_scatter(g):
            pltpu.async_copy(
                rows.at[pl.ds(0, 128)],
                acc.at[cdst.at[pl.ds(g * 128, 128)]], sem_s, add=True)

        def blk(b, woff):
            r0 = ebase + b * CB
            pltpu.sync_copy(dst_hbm.at[pl.ds(r0, CB)], didx)
            woff = _compact_block(None, didx, None, cdst, base, woff)
            ngroups = woff // 128

            def grp(g, _):
                issue_scatter(g)

                @pl.when(g >= sd)
                def _():
                    wait_scatter()

                return 0

            lax.fori_loop(0, ngroups, grp, 0)
            for k in range(sd):
                @pl.when(k < ngroups)
                def _():
                    wait_scatter()

            return _carry_remainder((cdst,), woff, ngroups)

        rem = lax.fori_loop(0, NBLK, blk, 0)
        for k in range(8):
            cdst[pl.ds(rem + k * 16, 16)] = trash

        @pl.when(rem > 0)
        def _():
            issue_scatter(0)
            wait_scatter()

        plsc.subcore_barrier()
        _drain_acc(out_hbm, acc, c, sid)

    return pl.kernel(
        body,
        out_type=jax.ShapeDtypeStruct((SROWS, d), jnp.float32),
        mesh=mesh,
        compiler_params=pltpu.CompilerParams(use_tc_tiling_on_sc=False,
                                             needs_layout_passes=False),
        scratch_types=[
            pltpu.VMEM((CB, 128), jnp.int32),
            pltpu.VMEM((CAP,), jnp.int32),
            pltpu.VMEM((232, d), jnp.float32),
            pltpu.VMEM_SHARED((ACC, d), jnp.float32),
            pltpu.SemaphoreType.DMA,
        ],
    )


_scat16 = _make_scatter(16, 8, 4)
_scat64 = _make_scatter(64, 2, 1)
_deg = _make_deg(4)


def _sblk(i):
    return ((i // 25) * 26 + (i % 25), 0)


def _tc_prep(deg16, x):
    def body(deg_ref, x_ref, g_ref):
        dinv = lax.rsqrt(deg_ref[:, 0:1] + 1.0)
        gx = x_ref[...] * dinv
        g_ref[...] = jnp.concatenate(
            [gx, jnp.zeros((RBLK, 16 - gx.shape[1]), jnp.float32)], axis=1)

    return pl.pallas_call(
        body,
        grid=(N // RBLK,),
        in_specs=[
            pl.BlockSpec((RBLK, 16), _sblk),
            pl.BlockSpec((RBLK, 5), lambda i: (i, 0)),
        ],
        out_specs=pl.BlockSpec((RBLK, 16), lambda i: (i, 0)),
        out_shape=jax.ShapeDtypeStruct((N, 16), jnp.float32),
    )(deg16, x)


def _tc_layer(s, g, deg16, w, b, relu_scale):
    din, dout = w.shape

    def body(s_ref, g_ref, deg_ref, w_ref, b_ref, o_ref):
        dinv = lax.rsqrt(deg_ref[:, 0:1] + 1.0)
        p = (s_ref[...] + g_ref[...]) * dinv
        h = jnp.dot(p, w_ref[...],
                    preferred_element_type=jnp.float32) + b_ref[...]
        if relu_scale:
            o_ref[...] = jnp.maximum(h, 0.0) * dinv
        else:
            o_ref[...] = h

    return pl.pallas_call(
        body,
        grid=(N // RBLK,),
        in_specs=[
            pl.BlockSpec((RBLK, din), _sblk),
            pl.BlockSpec((RBLK, din), lambda i: (i, 0)),
            pl.BlockSpec((RBLK, 16), _sblk),
            pl.BlockSpec((din, dout), lambda i: (0, 0)),
            pl.BlockSpec((1, dout), lambda i: (0, 0)),
        ],
        out_specs=pl.BlockSpec((RBLK, dout), lambda i: (i, 0)),
        out_shape=jax.ShapeDtypeStruct((N, dout), jnp.float32),
    )(s, g, deg16, w, b)


def kernel(x, edge_index, W1, b1, W2, b2, W3, b3):
    ei = edge_index.astype(jnp.int32)
    pad = EPAD - E
    srcp = jnp.concatenate(
        [ei[0], jnp.zeros((pad,), jnp.int32)]).reshape(EROWS, 128)
    dstp = jnp.concatenate(
        [ei[1], jnp.full((pad,), 2**30, jnp.int32)]).reshape(EROWS, 128)

    deg16 = _deg(dstp)
    g1 = _tc_prep(deg16, x)
    s1 = _scat16(g1, srcp, dstp)
    w1p = jnp.concatenate([W1, jnp.zeros((11, 64), jnp.float32)], axis=0)
    g2 = _tc_layer(s1, g1, deg16, w1p, b1.reshape(1, -1), True)
    s2 = _scat64(g2, srcp, dstp)
    g3 = _tc_layer(s2, g2, deg16, W2, b2.reshape(1, -1), True)
    s3 = _scat64(g3, srcp, dstp)
    return _tc_layer(s3, g3, deg16, W3, b3.reshape(1, -1), False)


# speedup vs baseline: 1.2422x; 1.0220x over previous
import jax
import jax.numpy as jnp
from jax import lax
from jax.experimental import pallas as pl
from jax.experimental.pallas import tpu as pltpu
from jax.experimental.pallas import tpu_sc as plsc

N = 50000
E = 800000
HALF = 25000
ACC = 26000
SROWS = 2 * ACC
NC, NS = 2, 16
EROWS = 6272
EPAD = EROWS * 128
RPT = EROWS // NS
CB = 8
NBLK = RPT // CB
CAP = 1280
TPT = 1624
RBLK = 1000


def _fill_rows(rows, nrows, d, vec):
    def zb(i, _):
        rows[i // (d // 16), pl.ds((i % (d // 16)) * 16, 16)] = vec
        return 0

    lax.fori_loop(0, nrows * (d // 16), zb, 0)


def _zero_acc(rows, acc, sid):
    for t in range(7):
        pltpu.sync_copy(rows.at[pl.ds(0, 232)],
                        acc.at[pl.ds(sid * TPT + t * 232, 232)])


def _drain_acc(out_hbm, acc, c, sid):
    pltpu.sync_copy(acc.at[pl.ds(sid * TPT, TPT)],
                    out_hbm.at[pl.ds(c * ACC + sid * TPT, TPT)])


def _compact_block(sidx, didx, csrc, cdst, base, woff):
    sentinel = jnp.int32(0x7FFFFFFF)

    def cv(i, w):
        j = i // 8
        k = i % 8
        dvec = didx[j, pl.ds(k * 16, 16)]
        l = dvec - base
        ok = (l >= 0) & (l < HALF)
        if sidx is not None:
            svec = sidx[j, pl.ds(k * 16, 16)]
            combo = jnp.where(ok, l * 65536 + svec, sentinel)
            sc = lax.sort(combo)
            cdst[pl.ds(w, 16)] = lax.shift_right_logical(sc, 16)
            csrc[pl.ds(w, 16)] = jnp.bitwise_and(sc, 0xFFFF)
        else:
            cdst[pl.ds(w, 16)] = lax.sort(jnp.where(ok, l, sentinel))
        return w + jnp.sum(ok.astype(jnp.int32))

    return lax.fori_loop(0, CB * 8, cv, woff)


def _carry_remainder(bufs, woff, ngroups):
    @pl.when(ngroups > 0)
    def _():
        off = ngroups * 128
        for buf in bufs:
            for k in range(8):
                v = buf[pl.ds(off + k * 16, 16)]
                buf[pl.ds(k * 16, 16)] = v

    return woff - ngroups * 128


def _make_scatter(d, nb, sd):
    mesh = plsc.VectorSubcoreMesh(core_axis_name="c", subcore_axis_name="s",
                                  num_cores=NC, num_subcores=NS)

    def body(g_hbm, src_hbm, dst_hbm, out_hbm, sidx, didx, csrc, cdst, rows,
             acc, sem_g, sem_s):
        c = lax.axis_index("c")
        sid = lax.axis_index("s")
        base = c * HALF
        trash = HALF + sid * 16 + lax.iota(jnp.int32, 16)
        ebase = sid * RPT
        _fill_rows(rows, 232, d, jnp.zeros((16,), jnp.float32))
        _zero_acc(rows, acc, sid)
        plsc.subcore_barrier()

        def wait_gather():
            pltpu.make_async_copy(
                g_hbm.at[csrc.at[pl.ds(0, 128)]],
                rows.at[pl.ds(0, 128)], sem_g).wait()

        def wait_scatter():
            pltpu.make_async_copy(
                rows.at[pl.ds(0, 128)],
                acc.at[cdst.at[pl.ds(0, 128)]], sem_s).wait()

        def issue_gather(g):
            pltpu.async_copy(
                g_hbm.at[csrc.at[pl.ds(g * 128, 128)]],
                rows.at[pl.ds(lax.rem(g, nb) * 128, 128)], sem_g)

        def issue_scatter(g):
            pltpu.async_copy(
                rows.at[pl.ds(lax.rem(g, nb) * 128, 128)],
                acc.at[cdst.at[pl.ds(g * 128, 128)]], sem_s, add=True)

        def stream_groups(ngroups):
            for k in range(nb - 1):
                @pl.when(k < ngroups)
                def _():
                    issue_gather(k)

            def grp(g, _):
                wait_gather()
                issue_scatter(g)

                @pl.when(g >= sd)
                def _():
                    wait_scatter()

                @pl.when(g + nb - 1 < ngroups)
                def _():
                    issue_gather(g + nb - 1)

                return 0

            lax.fori_loop(0, ngroups, grp, 0)
            for k in range(sd):
                @pl.when(k < ngroups)
                def _():
                    wait_scatter()

        def blk(b, woff):
            r0 = ebase + b * CB
            pltpu.sync_copy(src_hbm.at[pl.ds(r0, CB)], sidx)
            pltpu.sync_copy(dst_hbm.at[pl.ds(r0, CB)], didx)
            woff = _compact_block(sidx, didx, csrc, cdst, base, woff)
            ngroups = woff // 128
            stream_groups(ngroups)
            return _carry_remainder((csrc, cdst), woff, ngroups)

        rem = lax.fori_loop(0, NBLK, blk, 0)
        for k in range(8):
            cdst[pl.ds(rem + k * 16, 16)] = trash
            csrc[pl.ds(rem + k * 16, 16)] = jnp.zeros((16,), jnp.int32)

        @pl.when(rem > 0)
        def _():
            issue_gather(0)
            wait_gather()
            issue_scatter(0)
            wait_scatter()

        plsc.subcore_barrier()
        _drain_acc(out_hbm, acc, c, sid)

    return pl.kernel(
        body,
        out_type=jax.ShapeDtypeStruct((SROWS, d), jnp.float32),
        mesh=mesh,
        compiler_params=pltpu.CompilerParams(use_tc_tiling_on_sc=False,
                                             needs_layout_passes=False),
        scratch_types=[
            pltpu.VMEM((CB, 128), jnp.int32),
            pltpu.VMEM((CB, 128), jnp.int32),
            pltpu.VMEM((CAP,), jnp.int32),
            pltpu.VMEM((CAP,), jnp.int32),
            pltpu.VMEM((nb * 128, d), jnp.float32),
            pltpu.VMEM_SHARED((ACC, d), jnp.float32),
            pltpu.SemaphoreType.DMA,
            pltpu.SemaphoreType.DMA,
        ],
    )


def _make_deg(sd):
    mesh = plsc.VectorSubcoreMesh(core_axis_name="c", subcore_axis_name="s",
                                  num_cores=NC, num_subcores=NS)
    d = 16

    def body(dst_hbm, out_hbm, didx, rows, acc, sem_s):
        c = lax.axis_index("c")
        sid = lax.axis_index("s")
        base = c * HALF
        trash = HALF + sid * 16 + lax.iota(jnp.int32, 16)
        ebase = sid * RPT
        _fill_rows(rows, 232, d, jnp.zeros((16,), jnp.float32))
        _zero_acc(rows, acc, sid)
        _fill_rows(rows, 128, d,
                   jnp.where(lax.iota(jnp.int32, 16) == 0, 1.0, 0.0))
        plsc.subcore_barrier()

        def wait_scatter():
            pltpu.make_async_copy(
                rows.at[pl.ds(0, 128)],
                acc.at[didx.at[0, 0]], sem_s).wait()

        def blk(b, _):
            q = lax.rem(b, 2)
            r0 = ebase + b * CB
            pltpu.sync_copy(dst_hbm.at[pl.ds(r0, CB)], didx.at[q])

            def tb(i, _):
                j = i // 8
                k = i % 8
                dv = didx[q, j, pl.ds(k * 16, 16)]
                l = dv - base
                ok = (l >= 0) & (l < HALF)
                didx[q, j, pl.ds(k * 16, 16)] = jnp.where(ok, l, trash)
                return 0

            lax.fori_loop(0, CB * 8, tb, 0)
            for j in range(CB):
                pltpu.async_copy(rows.at[pl.ds(0, 128)],
                                 acc.at[didx.at[q, j]], sem_s, add=True)

                @pl.when(b * CB + j >= sd)
                def _():
                    wait_scatter()
            return 0

        lax.fori_loop(0, NBLK, blk, 0)
        for k in range(sd):
            wait_scatter()
        plsc.subcore_barrier()
        _drain_acc(out_hbm, acc, c, sid)

    return pl.kernel(
        body,
        out_type=jax.ShapeDtypeStruct((SROWS, d), jnp.float32),
        mesh=mesh,
        compiler_params=pltpu.CompilerParams(use_tc_tiling_on_sc=False,
                                             needs_layout_passes=False),
        scratch_types=[
            pltpu.VMEM((2, CB, 128), jnp.int32),
            pltpu.VMEM((232, d), jnp.float32),
            pltpu.VMEM_SHARED((ACC, d), jnp.float32),
            pltpu.SemaphoreType.DMA,
        ],
    )


_scat16 = _make_scatter(16, 8, 4)
_scat64 = _make_scatter(64, 2, 1)
_deg = _make_deg(4)


def _sblk(i):
    return ((i // 25) * 26 + (i % 25), 0)


def _tc_prep(deg16, x):
    def body(deg_ref, x_ref, g_ref):
        dinv = lax.rsqrt(deg_ref[:, 0:1] + 1.0)
        gx = x_ref[...] * dinv
        g_ref[...] = jnp.concatenate(
            [gx, jnp.zeros((RBLK, 16 - gx.shape[1]), jnp.float32)], axis=1)

    return pl.pallas_call(
        body,
        grid=(N // RBLK,),
        in_specs=[
            pl.BlockSpec((RBLK, 16), _sblk),
            pl.BlockSpec((RBLK, 5), lambda i: (i, 0)),
        ],
        out_specs=pl.BlockSpec((RBLK, 16), lambda i: (i, 0)),
        out_shape=jax.ShapeDtypeStruct((N, 16), jnp.float32),
    )(deg16, x)


def _tc_layer(s, g, deg16, w, b, relu_scale):
    din, dout = w.shape

    def body(s_ref, g_ref, deg_ref, w_ref, b_ref, o_ref):
        dinv = lax.rsqrt(deg_ref[:, 0:1] + 1.0)
        p = (s_ref[...] + g_ref[...]) * dinv
        h = jnp.dot(p, w_ref[...],
                    preferred_element_type=jnp.float32) + b_ref[...]
        if relu_scale:
            o_ref[...] = jnp.maximum(h, 0.0) * dinv
        else:
            o_ref[...] = h

    return pl.pallas_call(
        body,
        grid=(N // RBLK,),
        in_specs=[
            pl.BlockSpec((RBLK, din), _sblk),
            pl.BlockSpec((RBLK, din), lambda i: (i, 0)),
            pl.BlockSpec((RBLK, 16), _sblk),
            pl.BlockSpec((din, dout), lambda i: (0, 0)),
            pl.BlockSpec((1, dout), lambda i: (0, 0)),
        ],
        out_specs=pl.BlockSpec((RBLK, dout), lambda i: (i, 0)),
        out_shape=jax.ShapeDtypeStruct((N, dout), jnp.float32),
    )(s, g, deg16, w, b)


def kernel(x, edge_index, W1, b1, W2, b2, W3, b3):
    ei = edge_index.astype(jnp.int32)
    pad = EPAD - E
    srcp = jnp.concatenate(
        [ei[0], jnp.zeros((pad,), jnp.int32)]).reshape(EROWS, 128)
    dstp = jnp.concatenate(
        [ei[1], jnp.full((pad,), 2**30, jnp.int32)]).reshape(EROWS, 128)

    deg16 = _deg(dstp)
    g1 = _tc_prep(deg16, x)
    s1 = _scat16(g1, srcp, dstp)
    w1p = jnp.concatenate([W1, jnp.zeros((11, 64), jnp.float32)], axis=0)
    g2 = _tc_layer(s1, g1, deg16, w1p, b1.reshape(1, -1), True)
    s2 = _scat64(g2, srcp, dstp)
    g3 = _tc_layer(s2, g2, deg16, W2, b2.reshape(1, -1), True)
    s3 = _scat64(g3, srcp, dstp)
    return _tc_layer(s3, g3, deg16, W3, b3.reshape(1, -1), False)


# speedup vs baseline: 1.2426x; 1.0003x over previous
import jax
import jax.numpy as jnp
from jax import lax
from jax.experimental import pallas as pl
from jax.experimental.pallas import tpu as pltpu
from jax.experimental.pallas import tpu_sc as plsc

N = 50000
E = 800000
HALF = 25000
ACC = 26000
SROWS = 2 * ACC
NC, NS = 2, 16
EROWS = 6272
EPAD = EROWS * 128
RPT = EROWS // NS
CB = 8
NBLK = RPT // CB
CAP = 1280
TPT = 1624
RBLK = 1000


def _fill_rows(rows, nrows, d, vec):
    def zb(i, _):
        rows[i // (d // 16), pl.ds((i % (d // 16)) * 16, 16)] = vec
        return 0

    lax.fori_loop(0, nrows * (d // 16), zb, 0)


def _zero_acc(rows, acc, sid):
    for t in range(7):
        pltpu.sync_copy(rows.at[pl.ds(0, 232)],
                        acc.at[pl.ds(sid * TPT + t * 232, 232)])


def _drain_acc(out_hbm, acc, c, sid):
    pltpu.sync_copy(acc.at[pl.ds(sid * TPT, TPT)],
                    out_hbm.at[pl.ds(c * ACC + sid * TPT, TPT)])


def _compact_block(sidx, didx, csrc, cdst, base, woff):
    sentinel = jnp.int32(0x7FFFFFFF)

    def cv(i, w):
        j = i // 8
        k = i % 8
        dvec = didx[j, pl.ds(k * 16, 16)]
        l = dvec - base
        ok = (l >= 0) & (l < HALF)
        if sidx is not None:
            svec = sidx[j, pl.ds(k * 16, 16)]
            combo = jnp.where(ok, l * 65536 + svec, sentinel)
            sc = lax.sort(combo)
            cdst[pl.ds(w, 16)] = lax.shift_right_logical(sc, 16)
            csrc[pl.ds(w, 16)] = jnp.bitwise_and(sc, 0xFFFF)
        else:
            cdst[pl.ds(w, 16)] = lax.sort(jnp.where(ok, l, sentinel))
        return w + jnp.sum(ok.astype(jnp.int32))

    return lax.fori_loop(0, CB * 8, cv, woff)


def _carry_remainder(bufs, woff, ngroups):
    @pl.when(ngroups > 0)
    def _():
        off = ngroups * 128
        for buf in bufs:
            for k in range(8):
                v = buf[pl.ds(off + k * 16, 16)]
                buf[pl.ds(k * 16, 16)] = v

    return woff - ngroups * 128


def _make_scatter(d, nb, sd):
    mesh = plsc.VectorSubcoreMesh(core_axis_name="c", subcore_axis_name="s",
                                  num_cores=NC, num_subcores=NS)

    def body(g_hbm, src_hbm, dst_hbm, out_hbm, sidx, didx, csrc, cdst, rows,
             acc, sem_g, sem_s):
        c = lax.axis_index("c")
        sid = lax.axis_index("s")
        base = c * HALF
        trash = HALF + sid * 16 + lax.iota(jnp.int32, 16)
        ebase = sid * RPT
        _fill_rows(rows, 232, d, jnp.zeros((16,), jnp.float32))
        _zero_acc(rows, acc, sid)
        plsc.subcore_barrier()

        def wait_gather():
            pltpu.make_async_copy(
                g_hbm.at[csrc.at[pl.ds(0, 128)]],
                rows.at[pl.ds(0, 128)], sem_g).wait()

        def wait_scatter():
            pltpu.make_async_copy(
                rows.at[pl.ds(0, 128)],
                acc.at[cdst.at[pl.ds(0, 128)]], sem_s).wait()

        def issue_gather(g):
            pltpu.async_copy(
                g_hbm.at[csrc.at[pl.ds(g * 128, 128)]],
                rows.at[pl.ds(lax.rem(g, nb) * 128, 128)], sem_g)

        def issue_scatter(g):
            pltpu.async_copy(
                rows.at[pl.ds(lax.rem(g, nb) * 128, 128)],
                acc.at[cdst.at[pl.ds(g * 128, 128)]], sem_s, add=True)

        def stream_groups(ngroups):
            for k in range(nb - 1):
                @pl.when(k < ngroups)
                def _():
                    issue_gather(k)

            def grp(g, _):
                wait_gather()
                issue_scatter(g)

                @pl.when(g >= sd)
                def _():
                    wait_scatter()

                @pl.when(g + nb - 1 < ngroups)
                def _():
                    issue_gather(g + nb - 1)

                return 0

            lax.fori_loop(0, ngroups, grp, 0)
            for k in range(sd):
                @pl.when(k < ngroups)
                def _():
                    wait_scatter()

        def blk(b, woff):
            r0 = ebase + b * CB
            pltpu.sync_copy(src_hbm.at[pl.ds(r0, CB)], sidx)
            pltpu.sync_copy(dst_hbm.at[pl.ds(r0, CB)], didx)
            woff = _compact_block(sidx, didx, csrc, cdst, base, woff)
            ngroups = woff // 128
            stream_groups(ngroups)
            return _carry_remainder((csrc, cdst), woff, ngroups)

        rem = lax.fori_loop(0, NBLK, blk, 0)
        for k in range(8):
            cdst[pl.ds(rem + k * 16, 16)] = trash
            csrc[pl.ds(rem + k * 16, 16)] = jnp.zeros((16,), jnp.int32)

        @pl.when(rem > 0)
        def _():
            issue_gather(0)
            wait_gather()
            issue_scatter(0)
            wait_scatter()

        plsc.subcore_barrier()
        _drain_acc(out_hbm, acc, c, sid)

    return pl.kernel(
        body,
        out_type=jax.ShapeDtypeStruct((SROWS, d), jnp.float32),
        mesh=mesh,
        compiler_params=pltpu.CompilerParams(use_tc_tiling_on_sc=False,
                                             needs_layout_passes=False),
        scratch_types=[
            pltpu.VMEM((CB, 128), jnp.int32),
            pltpu.VMEM((CB, 128), jnp.int32),
            pltpu.VMEM((CAP,), jnp.int32),
            pltpu.VMEM((CAP,), jnp.int32),
            pltpu.VMEM((nb * 128, d), jnp.float32),
            pltpu.VMEM_SHARED((ACC, d), jnp.float32),
            pltpu.SemaphoreType.DMA,
            pltpu.SemaphoreType.DMA,
        ],
    )


def _make_deg(sd):
    mesh = plsc.VectorSubcoreMesh(core_axis_name="c", subcore_axis_name="s",
                                  num_cores=NC, num_subcores=NS)
    d = 16

    def body(dst_hbm, out_hbm, didx, rows, acc, sem_s):
        c = lax.axis_index("c")
        sid = lax.axis_index("s")
        base = c * HALF
        trash = HALF + sid * 16 + lax.iota(jnp.int32, 16)
        ebase = sid * RPT
        _fill_rows(rows, 232, d, jnp.zeros((16,), jnp.float32))
        _zero_acc(rows, acc, sid)
        _fill_rows(rows, 128, d,
                   jnp.where(lax.iota(jnp.int32, 16) == 0, 1.0, 0.0))
        plsc.subcore_barrier()

        def wait_scatter():
            pltpu.make_async_copy(
                rows.at[pl.ds(0, 128)],
                acc.at[didx.at[0, 0]], sem_s).wait()

        def blk(b, _):
            q = lax.rem(b, 2)
            r0 = ebase + b * CB
            pltpu.sync_copy(dst_hbm.at[pl.ds(r0, CB)], didx.at[q])

            def tb(i, _):
                j = i // 8
                k = i % 8
                dv = didx[q, j, pl.ds(k * 16, 16)]
                l = dv - base
                ok = (l >= 0) & (l < HALF)
                didx[q, j, pl.ds(k * 16, 16)] = jnp.where(ok, l, trash)
                return 0

            lax.fori_loop(0, CB * 8, tb, 0)
            for j in range(CB):
                pltpu.async_copy(rows.at[pl.ds(0, 128)],
                                 acc.at[didx.at[q, j]], sem_s, add=True)

                @pl.when(b * CB + j >= sd)
                def _():
                    wait_scatter()
            return 0

        lax.fori_loop(0, NBLK, blk, 0)
        for k in range(sd):
            wait_scatter()
        plsc.subcore_barrier()
        _drain_acc(out_hbm, acc, c, sid)

    return pl.kernel(
        body,
        out_type=jax.ShapeDtypeStruct((SROWS, d), jnp.float32),
        mesh=mesh,
        compiler_params=pltpu.CompilerParams(use_tc_tiling_on_sc=False,
                                             needs_layout_passes=False),
        scratch_types=[
            pltpu.VMEM((2, CB, 128), jnp.int32),
            pltpu.VMEM((232, d), jnp.float32),
            pltpu.VMEM_SHARED((ACC, d), jnp.float32),
            pltpu.SemaphoreType.DMA,
        ],
    )


_scat16 = _make_scatter(16, 9, 6)
_scat64 = _make_scatter(64, 2, 1)
_deg = _make_deg(4)


def _sblk(i):
    return ((i // 25) * 26 + (i % 25), 0)


def _tc_prep(deg16, x):
    def body(deg_ref, x_ref, g_ref):
        dinv = lax.rsqrt(deg_ref[:, 0:1] + 1.0)
        gx = x_ref[...] * dinv
        g_ref[...] = jnp.concatenate(
            [gx, jnp.zeros((RBLK, 16 - gx.shape[1]), jnp.float32)], axis=1)

    return pl.pallas_call(
        body,
        grid=(N // RBLK,),
        in_specs=[
            pl.BlockSpec((RBLK, 16), _sblk),
            pl.BlockSpec((RBLK, 5), lambda i: (i, 0)),
        ],
        out_specs=pl.BlockSpec((RBLK, 16), lambda i: (i, 0)),
        out_shape=jax.ShapeDtypeStruct((N, 16), jnp.float32),
    )(deg16, x)


def _tc_layer(s, g, deg16, w, b, relu_scale):
    din, dout = w.shape

    def body(s_ref, g_ref, deg_ref, w_ref, b_ref, o_ref):
        dinv = lax.rsqrt(deg_ref[:, 0:1] + 1.0)
        p = (s_ref[...] + g_ref[...]) * dinv
        h = jnp.dot(p, w_ref[...],
                    preferred_element_type=jnp.float32) + b_ref[...]
        if relu_scale:
            o_ref[...] = jnp.maximum(h, 0.0) * dinv
        else:
            o_ref[...] = h

    return pl.pallas_call(
        body,
        grid=(N // RBLK,),
        in_specs=[
            pl.BlockSpec((RBLK, din), _sblk),
            pl.BlockSpec((RBLK, din), lambda i: (i, 0)),
            pl.BlockSpec((RBLK, 16), _sblk),
            pl.BlockSpec((din, dout), lambda i: (0, 0)),
            pl.BlockSpec((1, dout), lambda i: (0, 0)),
        ],
        out_specs=pl.BlockSpec((RBLK, dout), lambda i: (i, 0)),
        out_shape=jax.ShapeDtypeStruct((N, dout), jnp.float32),
    )(s, g, deg16, w, b)


def kernel(x, edge_index, W1, b1, W2, b2, W3, b3):
    ei = edge_index.astype(jnp.int32)
    pad = EPAD - E
    srcp = jnp.concatenate(
        [ei[0], jnp.zeros((pad,), jnp.int32)]).reshape(EROWS, 128)
    dstp = jnp.concatenate(
        [ei[1], jnp.full((pad,), 2**30, jnp.int32)]).reshape(EROWS, 128)

    deg16 = _deg(dstp)
    g1 = _tc_prep(deg16, x)
    s1 = _scat16(g1, srcp, dstp)
    w1p = jnp.concatenate([W1, jnp.zeros((11, 64), jnp.float32)], axis=0)
    g2 = _tc_layer(s1, g1, deg16, w1p, b1.reshape(1, -1), True)
    s2 = _scat64(g2, srcp, dstp)
    g3 = _tc_layer(s2, g2, deg16, W2, b2.reshape(1, -1), True)
    s3 = _scat64(g3, srcp, dstp)
    return _tc_layer(s3, g3, deg16, W3, b3.reshape(1, -1), False)
